# initial kernel scaffold (unmeasured)
import jax
import jax.numpy as jnp
from jax import lax
from jax.experimental import pallas as pl
from jax.experimental.pallas import tpu as pltpu

N_DEV = 4
S = 1024
D = 2048
DC = 128
H = 16
DH = 128
DR = 32
SCALE = float((DH + DR) ** -0.5)


def _gather_kv_body(x_ref, wdkv_ref, wuk_ref, wuv_ref, wkr_ref,
                    xb_ref, k_ref, v_ref, kr_ref,
                    cbuf, kbuf, vbuf, send_sems, recv_sems):
    my = lax.axis_index("i")
    left = lax.rem(my + N_DEV - 1, N_DEV)
    right = lax.rem(my + 1, N_DEV)

    barrier = pltpu.get_barrier_semaphore()
    for nbr in (left, right):
        pl.semaphore_signal(barrier, inc=1, device_id=(nbr,),
                            device_id_type=pl.DeviceIdType.MESH)
    pl.semaphore_wait(barrier, 2)

    xb = x_ref[...].astype(jnp.bfloat16)
    xb_ref[...] = xb
    kr_ref[...] = jnp.dot(
        xb, wkr_ref[...].astype(jnp.bfloat16),
        preferred_element_type=jnp.float32).astype(jnp.bfloat16)

    cbuf[0] = jnp.dot(
        xb, wdkv_ref[...].astype(jnp.bfloat16),
        preferred_element_type=jnp.float32).astype(jnp.bfloat16)
    kbuf[0] = wuk_ref[...].astype(jnp.bfloat16)
    vbuf[0] = wuv_ref[...].astype(jnp.bfloat16)

    for h in range(N_DEV - 1):
        rdmas = []
        for t, buf in enumerate((cbuf, kbuf, vbuf)):
            rdma = pltpu.make_async_remote_copy(
                src_ref=buf.at[h],
                dst_ref=buf.at[h + 1],
                send_sem=send_sems.at[t, h],
                recv_sem=recv_sems.at[t, h],
                device_id=(right,),
                device_id_type=pl.DeviceIdType.MESH,
            )
            rdma.start()
            rdmas.append(rdma)
        for rdma in rdmas:
            rdma.wait()

    k_acc = jnp.dot(cbuf[0], kbuf[0], preferred_element_type=jnp.float32)
    v_acc = jnp.dot(cbuf[0], vbuf[0], preferred_element_type=jnp.float32)
    for j in range(1, N_DEV):
        k_acc += jnp.dot(cbuf[j], kbuf[j], preferred_element_type=jnp.float32)
        v_acc += jnp.dot(cbuf[j], vbuf[j], preferred_element_type=jnp.float32)
    k_ref[...] = k_acc.astype(jnp.bfloat16)
    v_ref[...] = v_acc.astype(jnp.bfloat16)


def _attn_body(xb_ref, wq_ref, wqr_ref, k_ref, v_ref, kr_ref, o_ref):
    xb = xb_ref[...]
    q = jnp.dot(xb, wq_ref[...].astype(jnp.bfloat16),
                preferred_element_type=jnp.float32).astype(jnp.bfloat16)
    qr = jnp.dot(xb, wqr_ref[...].astype(jnp.bfloat16),
                 preferred_element_type=jnp.float32).astype(jnp.bfloat16)
    s = lax.dot_general(q, k_ref[...], (((1,), (1,)), ((), ())),
                        preferred_element_type=jnp.float32)
    s = s + lax.dot_general(qr, kr_ref[...], (((1,), (1,)), ((), ())),
                            preferred_element_type=jnp.float32)
    s = s * SCALE
    m = jnp.max(s, axis=1, keepdims=True)
    p = jnp.exp(s - m)
    p = p / jnp.sum(p, axis=1, keepdims=True)
    o_ref[...] = jnp.dot(p.astype(jnp.bfloat16), v_ref[...],
                         preferred_element_type=jnp.float32).astype(jnp.bfloat16)


def _proj_body(o_ref, wo_ref, out_ref):
    out_ref[...] = jnp.dot(o_ref[...], wo_ref[...].astype(jnp.bfloat16),
                           preferred_element_type=jnp.float32)


def kernel(x, Wdkv, Wuk, Wuv, Wq, Wqr, Wkr, Wo):
    x2 = x[0]

    xb, K, V, Kr = pl.pallas_call(
        _gather_kv_body,
        out_shape=[
            jax.ShapeDtypeStruct((S, D), jnp.bfloat16),
            jax.ShapeDtypeStruct((S, D), jnp.bfloat16),
            jax.ShapeDtypeStruct((S, D), jnp.bfloat16),
            jax.ShapeDtypeStruct((S, DR), jnp.bfloat16),
        ],
        in_specs=[pl.BlockSpec(memory_space=pltpu.VMEM)] * 5,
        out_specs=[pl.BlockSpec(memory_space=pltpu.VMEM)] * 4,
        scratch_shapes=[
            pltpu.VMEM((N_DEV, S, DC), jnp.bfloat16),
            pltpu.VMEM((N_DEV, DC, D), jnp.bfloat16),
            pltpu.VMEM((N_DEV, DC, D), jnp.bfloat16),
            pltpu.SemaphoreType.DMA((3, N_DEV - 1)),
            pltpu.SemaphoreType.DMA((3, N_DEV - 1)),
        ],
        compiler_params=pltpu.CompilerParams(collective_id=0),
    )(x2, Wdkv, Wuk, Wuv, Wkr)

    O = pl.pallas_call(
        _attn_body,
        grid=(H,),
        out_shape=jax.ShapeDtypeStruct((S, D), jnp.bfloat16),
        in_specs=[
            pl.BlockSpec((S, D), lambda h: (0, 0)),
            pl.BlockSpec((D, DH), lambda h: (0, h)),
            pl.BlockSpec((D, DR), lambda h: (0, h)),
            pl.BlockSpec((S, DH), lambda h: (0, h)),
            pl.BlockSpec((S, DH), lambda h: (0, h)),
            pl.BlockSpec((S, DR), lambda h: (0, 0)),
        ],
        out_specs=pl.BlockSpec((S, DH), lambda h: (0, h)),
    )(xb, Wq, Wqr, K, V, Kr)

    out = pl.pallas_call(
        _proj_body,
        grid=(4,),
        out_shape=jax.ShapeDtypeStruct((S, D), jnp.float32),
        in_specs=[
            pl.BlockSpec((S, D), lambda j: (0, 0)),
            pl.BlockSpec((D, D // 4), lambda j: (0, j)),
        ],
        out_specs=pl.BlockSpec((S, D // 4), lambda j: (0, j)),
    )(O, Wo)

    return out[None]


# baseline (device time: 196165 ns/iter reference)
import jax
import jax.numpy as jnp
from jax import lax
from jax.experimental import pallas as pl
from jax.experimental.pallas import tpu as pltpu

N_DEV = 4
S = 1024
D = 2048
DC = 128
H = 16
DH = 128
DR = 32
SCALE = float((DH + DR) ** -0.5)


def _gather_kv_body(x_ref, wdkv_ref, wuk_ref, wuv_ref, wkr_ref,
                    xb_ref, k_ref, v_ref, kr_ref,
                    cbuf, kbuf, vbuf, send_sems, recv_sems):
    my = lax.axis_index("i")
    left = lax.rem(my + N_DEV - 1, N_DEV)
    right = lax.rem(my + 1, N_DEV)

    barrier = pltpu.get_barrier_semaphore()
    for nbr in (left, right):
        pl.semaphore_signal(barrier, inc=1, device_id=(nbr,),
                            device_id_type=pl.DeviceIdType.MESH)
    pl.semaphore_wait(barrier, 2)

    xb = x_ref[...].astype(jnp.bfloat16)
    xb_ref[...] = xb
    kr_ref[...] = jnp.dot(
        xb, wkr_ref[...].astype(jnp.bfloat16),
        preferred_element_type=jnp.float32).astype(jnp.bfloat16)

    cbuf[0] = jnp.dot(
        xb, wdkv_ref[...].astype(jnp.bfloat16),
        preferred_element_type=jnp.float32).astype(jnp.bfloat16)
    kbuf[0] = wuk_ref[...].astype(jnp.bfloat16)
    vbuf[0] = wuv_ref[...].astype(jnp.bfloat16)

    for h in range(N_DEV - 1):
        rdmas = []
        for t, buf in enumerate((cbuf, kbuf, vbuf)):
            rdma = pltpu.make_async_remote_copy(
                src_ref=buf.at[h],
                dst_ref=buf.at[h + 1],
                send_sem=send_sems.at[t, h],
                recv_sem=recv_sems.at[t, h],
                device_id=(right,),
                device_id_type=pl.DeviceIdType.MESH,
            )
            rdma.start()
            rdmas.append(rdma)
        for rdma in rdmas:
            rdma.wait()

    k_acc = jnp.dot(cbuf[0], kbuf[0], preferred_element_type=jnp.float32)
    v_acc = jnp.dot(cbuf[0], vbuf[0], preferred_element_type=jnp.float32)
    for j in range(1, N_DEV):
        k_acc += jnp.dot(cbuf[j], kbuf[j], preferred_element_type=jnp.float32)
        v_acc += jnp.dot(cbuf[j], vbuf[j], preferred_element_type=jnp.float32)
    k_ref[...] = k_acc.astype(jnp.bfloat16)
    v_ref[...] = v_acc.astype(jnp.bfloat16)


def _attn_body(xb_ref, wq_ref, wqr_ref, k_ref, v_ref, kr_ref, o_ref):
    xb = xb_ref[...]
    q = jnp.dot(xb, wq_ref[...].astype(jnp.bfloat16),
                preferred_element_type=jnp.float32).astype(jnp.bfloat16)
    qr = jnp.dot(xb, wqr_ref[0].astype(jnp.bfloat16),
                 preferred_element_type=jnp.float32).astype(jnp.bfloat16)
    s = lax.dot_general(q, k_ref[...], (((1,), (1,)), ((), ())),
                        preferred_element_type=jnp.float32)
    s = s + lax.dot_general(qr, kr_ref[...], (((1,), (1,)), ((), ())),
                            preferred_element_type=jnp.float32)
    s = s * SCALE
    m = jnp.max(s, axis=1, keepdims=True)
    p = jnp.exp(s - m)
    p = p / jnp.sum(p, axis=1, keepdims=True)
    o_ref[...] = jnp.dot(p.astype(jnp.bfloat16), v_ref[...],
                         preferred_element_type=jnp.float32).astype(jnp.bfloat16)


def _proj_body(o_ref, wo_ref, out_ref):
    out_ref[...] = jnp.dot(o_ref[...], wo_ref[...].astype(jnp.bfloat16),
                           preferred_element_type=jnp.float32)


def kernel(x, Wdkv, Wuk, Wuv, Wq, Wqr, Wkr, Wo):
    x2 = x[0]

    xb, K, V, Kr = pl.pallas_call(
        _gather_kv_body,
        out_shape=[
            jax.ShapeDtypeStruct((S, D), jnp.bfloat16),
            jax.ShapeDtypeStruct((S, D), jnp.bfloat16),
            jax.ShapeDtypeStruct((S, D), jnp.bfloat16),
            jax.ShapeDtypeStruct((S, DR), jnp.bfloat16),
        ],
        in_specs=[pl.BlockSpec(memory_space=pltpu.VMEM)] * 5,
        out_specs=[pl.BlockSpec(memory_space=pltpu.VMEM)] * 4,
        scratch_shapes=[
            pltpu.VMEM((N_DEV, S, DC), jnp.bfloat16),
            pltpu.VMEM((N_DEV, DC, D), jnp.bfloat16),
            pltpu.VMEM((N_DEV, DC, D), jnp.bfloat16),
            pltpu.SemaphoreType.DMA((3, N_DEV - 1)),
            pltpu.SemaphoreType.DMA((3, N_DEV - 1)),
        ],
        compiler_params=pltpu.CompilerParams(collective_id=0),
    )(x2, Wdkv, Wuk, Wuv, Wkr)

    Wqr3 = jnp.transpose(Wqr.reshape(D, H, DR), (1, 0, 2))

    O = pl.pallas_call(
        _attn_body,
        grid=(H,),
        out_shape=jax.ShapeDtypeStruct((S, D), jnp.bfloat16),
        in_specs=[
            pl.BlockSpec((S, D), lambda h: (0, 0)),
            pl.BlockSpec((D, DH), lambda h: (0, h)),
            pl.BlockSpec((1, D, DR), lambda h: (h, 0, 0)),
            pl.BlockSpec((S, DH), lambda h: (0, h)),
            pl.BlockSpec((S, DH), lambda h: (0, h)),
            pl.BlockSpec((S, DR), lambda h: (0, 0)),
        ],
        out_specs=pl.BlockSpec((S, DH), lambda h: (0, h)),
    )(xb, Wq, Wqr3, K, V, Kr)

    out = pl.pallas_call(
        _proj_body,
        grid=(4,),
        out_shape=jax.ShapeDtypeStruct((S, D), jnp.float32),
        in_specs=[
            pl.BlockSpec((S, D), lambda j: (0, 0)),
            pl.BlockSpec((D, D // 4), lambda j: (0, j)),
        ],
        out_specs=pl.BlockSpec((S, D // 4), lambda j: (0, j)),
    )(O, Wo)

    return out[None]


# device time: 88524 ns/iter; 2.2160x vs baseline; 2.2160x over previous
import functools

import jax
import jax.numpy as jnp
from jax import lax
from jax.experimental import pallas as pl
from jax.experimental.pallas import tpu as pltpu

N_DEV = 4
S = 1024
D = 2048
DC = 128
H = 16
HP = H // N_DEV
DH = 128
DR = 32
HD = HP * DH
SCALE = float((DH + DR) ** -0.5)


def _gather_body(xb_ref, wdkv_ref, wuk_ref, wuv_ref, wkr_ref,
                 k_ref, v_ref, kr_ref,
                 cbuf, wkbuf, wvbuf, send_sems, recv_sems, loc_sems):
    my = lax.axis_index("i")

    barrier = pltpu.get_barrier_semaphore()
    for k in range(1, N_DEV):
        pl.semaphore_signal(barrier, inc=1,
                            device_id=(lax.rem(my + k, N_DEV),),
                            device_id_type=pl.DeviceIdType.MESH)
    pl.semaphore_wait(barrier, N_DEV - 1)

    xb = xb_ref[...]
    kr_ref[...] = jnp.dot(xb, wkr_ref[...],
                          preferred_element_type=jnp.float32
                          ).astype(jnp.bfloat16)
    cbuf[0] = jnp.dot(xb, wdkv_ref[...],
                      preferred_element_type=jnp.float32
                      ).astype(jnp.bfloat16)

    own_k = pltpu.make_async_copy(
        wuk_ref.at[:, pl.ds(my * HD, HD)], wkbuf.at[0], loc_sems.at[0])
    own_v = pltpu.make_async_copy(
        wuv_ref.at[:, pl.ds(my * HD, HD)], wvbuf.at[0], loc_sems.at[1])
    own_k.start()
    own_v.start()

    def _sends(start):
        rdmas = []
        for k in range(1, N_DEV):
            d = lax.rem(my + k, N_DEV)
            slot = N_DEV - k
            for t, (src, dst) in enumerate((
                (cbuf.at[0], cbuf.at[slot]),
                (wuk_ref.at[:, pl.ds(d * HD, HD)], wkbuf.at[slot]),
                (wuv_ref.at[:, pl.ds(d * HD, HD)], wvbuf.at[slot]),
            )):
                rdma = pltpu.make_async_remote_copy(
                    src_ref=src, dst_ref=dst,
                    send_sem=send_sems.at[slot - 1, t],
                    recv_sem=recv_sems.at[slot - 1, t],
                    device_id=(d,), device_id_type=pl.DeviceIdType.MESH,
                )
                if start:
                    rdma.start()
                rdmas.append(rdma)
        return rdmas

    rdmas = _sends(start=True)
    own_k.wait()
    own_v.wait()
    for rdma in rdmas:
        rdma.wait()

    k_acc = jnp.dot(cbuf[0], wkbuf[0], preferred_element_type=jnp.float32)
    v_acc = jnp.dot(cbuf[0], wvbuf[0], preferred_element_type=jnp.float32)
    for j in range(1, N_DEV):
        k_acc += jnp.dot(cbuf[j], wkbuf[j], preferred_element_type=jnp.float32)
        v_acc += jnp.dot(cbuf[j], wvbuf[j], preferred_element_type=jnp.float32)
    k_ref[...] = k_acc.astype(jnp.bfloat16)
    v_ref[...] = v_acc.astype(jnp.bfloat16)


def _attn_body(xb_ref, wq_ref, wqr_ref, k_ref, v_ref, kr_ref,
               o_ref, send_sems, recv_sems):
    my = lax.axis_index("i")
    h = pl.program_id(0)

    @pl.when(h == 0)
    def _():
        barrier = pltpu.get_barrier_semaphore()
        for k in range(1, N_DEV):
            pl.semaphore_signal(barrier, inc=1,
                                device_id=(lax.rem(my + k, N_DEV),),
                                device_id_type=pl.DeviceIdType.MESH)
        pl.semaphore_wait(barrier, N_DEV - 1)

    xb = xb_ref[...]
    q = jnp.dot(xb, wq_ref[...],
                preferred_element_type=jnp.float32).astype(jnp.bfloat16)
    qr = jnp.dot(xb, wqr_ref[0],
                 preferred_element_type=jnp.float32).astype(jnp.bfloat16)
    qcat = jnp.concatenate([q, qr], axis=1)
    kcat = jnp.concatenate([k_ref[...], kr_ref[...]], axis=1)
    s = lax.dot_general(qcat, kcat, (((1,), (1,)), ((), ())),
                        preferred_element_type=jnp.float32)
    p = jnp.exp(s * SCALE)
    denom = jnp.sum(p, axis=1, keepdims=True)
    o = jnp.dot(p.astype(jnp.bfloat16), v_ref[...],
                preferred_element_type=jnp.float32)
    o = o * (1.0 / denom)

    def _hop_rdmas(hh, start):
        rdmas = []
        for k in range(1, N_DEV):
            d = lax.rem(my + k, N_DEV)
            slot = N_DEV - k
            rdma = pltpu.make_async_remote_copy(
                src_ref=o_ref.at[:, pl.ds(hh * DH, DH)],
                dst_ref=o_ref.at[:, pl.ds(slot * HD + hh * DH, DH)],
                send_sem=send_sems.at[slot - 1, hh],
                recv_sem=recv_sems.at[slot - 1, hh],
                device_id=(d,), device_id_type=pl.DeviceIdType.MESH,
            )
            if start:
                rdma.start()
            rdmas.append(rdma)
        return rdmas

    o_ref[:, pl.ds(h * DH, DH)] = o.astype(jnp.bfloat16)
    _hop_rdmas(h, start=True)

    @pl.when(h == HP - 1)
    def _():
        for hh in range(HP):
            for rdma in _hop_rdmas(hh, start=False):
                rdma.wait()


def _proj_body(o_ref, wo_ref, out_ref):
    out_ref[...] = jnp.dot(o_ref[...], wo_ref[...],
                           preferred_element_type=jnp.float32)


def kernel(x, Wdkv, Wuk, Wuv, Wq, Wqr, Wkr, Wo):
    my = lax.axis_index("i")
    bf = jnp.bfloat16

    xb = x[0].astype(bf)
    wq_own = lax.dynamic_slice(Wq.astype(bf), (0, my * HD), (D, HD))
    wqr3 = jnp.transpose(Wqr.astype(bf).reshape(D, H, DR), (1, 0, 2))
    wqr_own = lax.dynamic_slice(wqr3, (my * HP, 0, 0), (HP, D, DR))

    K, V, Kr = pl.pallas_call(
        _gather_body,
        out_shape=[
            jax.ShapeDtypeStruct((S, HD), bf),
            jax.ShapeDtypeStruct((S, HD), bf),
            jax.ShapeDtypeStruct((S, DR), bf),
        ],
        in_specs=[pl.BlockSpec(memory_space=pltpu.VMEM)] * 5,
        out_specs=[pl.BlockSpec(memory_space=pltpu.VMEM)] * 3,
        scratch_shapes=[
            pltpu.VMEM((N_DEV, S, DC), bf),
            pltpu.VMEM((N_DEV, DC, HD), bf),
            pltpu.VMEM((N_DEV, DC, HD), bf),
            pltpu.SemaphoreType.DMA((N_DEV - 1, 3)),
            pltpu.SemaphoreType.DMA((N_DEV - 1, 3)),
            pltpu.SemaphoreType.DMA((2,)),
        ],
        compiler_params=pltpu.CompilerParams(collective_id=0),
    )(xb, Wdkv.astype(bf), Wuk.astype(bf), Wuv.astype(bf), Wkr.astype(bf))

    O_slot = pl.pallas_call(
        _attn_body,
        grid=(HP,),
        out_shape=jax.ShapeDtypeStruct((S, D), bf),
        in_specs=[
            pl.BlockSpec((S, D), lambda h: (0, 0)),
            pl.BlockSpec((D, DH), lambda h: (0, h)),
            pl.BlockSpec((1, D, DR), lambda h: (h, 0, 0)),
            pl.BlockSpec((S, DH), lambda h: (0, h)),
            pl.BlockSpec((S, DH), lambda h: (0, h)),
            pl.BlockSpec((S, DR), lambda h: (0, 0)),
        ],
        out_specs=pl.BlockSpec((S, D), lambda h: (0, 0)),
        scratch_shapes=[
            pltpu.SemaphoreType.DMA((N_DEV - 1, HP)),
            pltpu.SemaphoreType.DMA((N_DEV - 1, HP)),
        ],
        compiler_params=pltpu.CompilerParams(collective_id=1),
    )(xb, wq_own, wqr_own, K, V, Kr)

    O = jnp.roll(O_slot, my * HD, axis=1)

    out = pl.pallas_call(
        _proj_body,
        grid=(4,),
        out_shape=jax.ShapeDtypeStruct((S, D), jnp.float32),
        in_specs=[
            pl.BlockSpec((S, D), lambda j: (0, 0)),
            pl.BlockSpec((D, D // 4), lambda j: (0, j)),
        ],
        out_specs=pl.BlockSpec((S, D // 4), lambda j: (0, j)),
    )(O, Wo.astype(bf))

    return out[None]


# device time: 81391 ns/iter; 2.4102x vs baseline; 1.0876x over previous
import jax
import jax.numpy as jnp
from jax import lax
from jax.experimental import pallas as pl
from jax.experimental.pallas import tpu as pltpu

N_DEV = 4
S = 1024
D = 2048
DC = 128
H = 16
HP = H // N_DEV
DH = 128
DR = 32
HD = HP * DH
SCALE = float((DH + DR) ** -0.5)
BF = jnp.bfloat16
F32 = jnp.float32


def _gather_body(x_ref, wdkv_ref, wuk_ref, wuv_ref, wkr_ref,
                 xb_ref, k_ref, v_ref, kr_ref,
                 cbuf, wkb, wvb, wkbuf, wvbuf,
                 send_sems, recv_sems, loc_sems):
    my = lax.axis_index("i")

    xb = x_ref[...].astype(BF)
    xb_ref[...] = xb
    kr_ref[...] = jnp.dot(xb, wkr_ref[...].astype(BF),
                          preferred_element_type=F32).astype(BF)
    cbuf[0] = jnp.dot(xb, wdkv_ref[...].astype(BF),
                      preferred_element_type=F32).astype(BF)
    wkb[...] = wuk_ref[...].astype(BF)
    wvb[...] = wuv_ref[...].astype(BF)

    barrier = pltpu.get_barrier_semaphore()
    for k in range(1, N_DEV):
        pl.semaphore_signal(barrier, inc=1,
                            device_id=(lax.rem(my + k, N_DEV),),
                            device_id_type=pl.DeviceIdType.MESH)
    pl.semaphore_wait(barrier, N_DEV - 1)

    own_k = pltpu.make_async_copy(
        wkb.at[:, pl.ds(my * HD, HD)], wkbuf.at[0], loc_sems.at[0])
    own_v = pltpu.make_async_copy(
        wvb.at[:, pl.ds(my * HD, HD)], wvbuf.at[0], loc_sems.at[1])
    own_k.start()
    own_v.start()

    def _sends(start):
        rdmas = []
        for k in range(1, N_DEV):
            d = lax.rem(my + k, N_DEV)
            slot = N_DEV - k
            for t, (src, dst) in enumerate((
                (cbuf.at[0], cbuf.at[slot]),
                (wkb.at[:, pl.ds(d * HD, HD)], wkbuf.at[slot]),
                (wvb.at[:, pl.ds(d * HD, HD)], wvbuf.at[slot]),
            )):
                rdma = pltpu.make_async_remote_copy(
                    src_ref=src, dst_ref=dst,
                    send_sem=send_sems.at[slot - 1, t],
                    recv_sem=recv_sems.at[slot - 1, t],
                    device_id=(d,), device_id_type=pl.DeviceIdType.MESH,
                )
                if start:
                    rdma.start()
                rdmas.append(rdma)
        return rdmas

    rdmas = _sends(start=True)
    own_k.wait()
    own_v.wait()
    for rdma in rdmas:
        rdma.wait()

    k_acc = jnp.dot(cbuf[0], wkbuf[0], preferred_element_type=F32)
    v_acc = jnp.dot(cbuf[0], wvbuf[0], preferred_element_type=F32)
    for j in range(1, N_DEV):
        k_acc += jnp.dot(cbuf[j], wkbuf[j], preferred_element_type=F32)
        v_acc += jnp.dot(cbuf[j], wvbuf[j], preferred_element_type=F32)
    k_ref[...] = k_acc.astype(BF)
    v_ref[...] = v_acc.astype(BF)


def _attn_body(xb_ref, wq_ref, wqr_ref, k_ref, v_ref, kr_ref,
               o_ref, obuf, send_sems, recv_sems, loc_sems):
    my = lax.axis_index("i")
    h = pl.program_id(0)

    @pl.when(h == 0)
    def _():
        barrier = pltpu.get_barrier_semaphore()
        for k in range(1, N_DEV):
            pl.semaphore_signal(barrier, inc=1,
                                device_id=(lax.rem(my + k, N_DEV),),
                                device_id_type=pl.DeviceIdType.MESH)
        pl.semaphore_wait(barrier, N_DEV - 1)

    xb = xb_ref[...]
    q = jnp.dot(xb, wq_ref[...], preferred_element_type=F32).astype(BF)
    qr = jnp.dot(xb, wqr_ref[0], preferred_element_type=F32).astype(BF)
    qcat = jnp.concatenate([q, qr], axis=1)
    kcat = jnp.concatenate([k_ref[...], kr_ref[...]], axis=1)
    s = lax.dot_general(qcat, kcat, (((1,), (1,)), ((), ())),
                        preferred_element_type=F32)
    p = jnp.exp(s * SCALE)
    denom = jnp.sum(p, axis=1, keepdims=True)
    o = jnp.dot(p.astype(BF), v_ref[...], preferred_element_type=F32)
    obuf[h] = (o * (1.0 / denom)).astype(BF)

    def _hop_rdmas(hh, start):
        col = (my * HP + hh) * DH
        rdmas = [pltpu.make_async_copy(
            obuf.at[hh], o_ref.at[:, pl.ds(col, DH)], loc_sems.at[hh])]
        for k in range(1, N_DEV):
            d = lax.rem(my + k, N_DEV)
            rdmas.append(pltpu.make_async_remote_copy(
                src_ref=obuf.at[hh],
                dst_ref=o_ref.at[:, pl.ds(col, DH)],
                send_sem=send_sems.at[N_DEV - 1 - k, hh],
                recv_sem=recv_sems.at[N_DEV - 1 - k, hh],
                device_id=(d,), device_id_type=pl.DeviceIdType.MESH,
            ))
        if start:
            for rdma in rdmas:
                rdma.start()
        return rdmas

    _hop_rdmas(h, start=True)

    @pl.when(h == HP - 1)
    def _():
        for hh in range(HP):
            rdmas = _hop_rdmas(hh, start=False)
            rdmas[0].wait()
            for rdma in rdmas[1:]:
                rdma.wait()


def _proj_body(o_ref, wo_ref, out_ref):
    out_ref[...] = jnp.dot(o_ref[...], wo_ref[...].astype(BF),
                           preferred_element_type=F32)


def kernel(x, Wdkv, Wuk, Wuv, Wq, Wqr, Wkr, Wo):
    my = lax.axis_index("i")

    wq_own = lax.dynamic_slice(Wq.astype(BF), (0, my * HD), (D, HD))
    wqr3 = jnp.transpose(Wqr.astype(BF).reshape(D, H, DR), (1, 0, 2))
    wqr_own = lax.dynamic_slice(wqr3, (my * HP, 0, 0), (HP, D, DR))

    xb, K, V, Kr = pl.pallas_call(
        _gather_body,
        out_shape=[
            jax.ShapeDtypeStruct((S, D), BF),
            jax.ShapeDtypeStruct((S, HD), BF),
            jax.ShapeDtypeStruct((S, HD), BF),
            jax.ShapeDtypeStruct((S, DR), BF),
        ],
        in_specs=[pl.BlockSpec(memory_space=pltpu.VMEM)] * 5,
        out_specs=[pl.BlockSpec(memory_space=pltpu.VMEM)] * 4,
        scratch_shapes=[
            pltpu.VMEM((N_DEV, S, DC), BF),
            pltpu.VMEM((DC, D), BF),
            pltpu.VMEM((DC, D), BF),
            pltpu.VMEM((N_DEV, DC, HD), BF),
            pltpu.VMEM((N_DEV, DC, HD), BF),
            pltpu.SemaphoreType.DMA((N_DEV - 1, 3)),
            pltpu.SemaphoreType.DMA((N_DEV - 1, 3)),
            pltpu.SemaphoreType.DMA((2,)),
        ],
        compiler_params=pltpu.CompilerParams(collective_id=0),
    )(x[0], Wdkv, Wuk, Wuv, Wkr)

    O = pl.pallas_call(
        _attn_body,
        grid=(HP,),
        out_shape=jax.ShapeDtypeStruct((S, D), BF),
        in_specs=[
            pl.BlockSpec((S, D), lambda h: (0, 0)),
            pl.BlockSpec((D, DH), lambda h: (0, h)),
            pl.BlockSpec((1, D, DR), lambda h: (h, 0, 0)),
            pl.BlockSpec((S, DH), lambda h: (0, h)),
            pl.BlockSpec((S, DH), lambda h: (0, h)),
            pl.BlockSpec((S, DR), lambda h: (0, 0)),
        ],
        out_specs=pl.BlockSpec((S, D), lambda h: (0, 0)),
        scratch_shapes=[
            pltpu.VMEM((HP, S, DH), BF),
            pltpu.SemaphoreType.DMA((N_DEV - 1, HP)),
            pltpu.SemaphoreType.DMA((N_DEV - 1, HP)),
            pltpu.SemaphoreType.DMA((HP,)),
        ],
        compiler_params=pltpu.CompilerParams(collective_id=1),
    )(xb, wq_own, wqr_own, K, V, Kr)

    out = pl.pallas_call(
        _proj_body,
        grid=(4,),
        out_shape=jax.ShapeDtypeStruct((S, D), F32),
        in_specs=[
            pl.BlockSpec((S, D), lambda j: (0, 0)),
            pl.BlockSpec((D, D // 4), lambda j: (0, j)),
        ],
        out_specs=pl.BlockSpec((S, D // 4), lambda j: (0, j)),
    )(O, Wo)

    return out[None]


# device time: 76189 ns/iter; 2.5747x vs baseline; 1.0683x over previous
import jax
import jax.numpy as jnp
from jax import lax
from jax.experimental import pallas as pl
from jax.experimental.pallas import tpu as pltpu

N_DEV = 4
S = 1024
D = 2048
DC = 128
H = 16
HP = H // N_DEV
DH = 128
DR = 32
HD = HP * DH
SCALE = float((DH + DR) ** -0.5)
BF = jnp.bfloat16
F32 = jnp.float32


def _gather_body(x_ref, wdkv_ref, wuk_ref, wuv_ref, wkr_ref,
                 wq_ref, wqr_ref,
                 q_ref, qr_ref, k_ref, v_ref, kr_ref,
                 cbuf, wkb, wvb, wkbuf, wvbuf,
                 send_sems, recv_sems, loc_sems):
    my = lax.axis_index("i")

    xb = x_ref[0].astype(BF)
    kr_ref[...] = jnp.dot(xb, wkr_ref[...].astype(BF),
                          preferred_element_type=F32).astype(BF)
    cbuf[0] = jnp.dot(xb, wdkv_ref[...].astype(BF),
                      preferred_element_type=F32).astype(BF)
    wkb[...] = wuk_ref[...].astype(BF)
    wvb[...] = wuv_ref[...].astype(BF)

    barrier = pltpu.get_barrier_semaphore()
    for k in range(1, N_DEV):
        pl.semaphore_signal(barrier, inc=1,
                            device_id=(lax.rem(my + k, N_DEV),),
                            device_id_type=pl.DeviceIdType.MESH)
    pl.semaphore_wait(barrier, N_DEV - 1)

    own_k = pltpu.make_async_copy(
        wkb.at[:, pl.ds(my * HD, HD)], wkbuf.at[0], loc_sems.at[0])
    own_v = pltpu.make_async_copy(
        wvb.at[:, pl.ds(my * HD, HD)], wvbuf.at[0], loc_sems.at[1])
    own_k.start()
    own_v.start()

    def _sends(start):
        rdmas = []
        for k in range(1, N_DEV):
            d = lax.rem(my + k, N_DEV)
            slot = N_DEV - k
            for t, (src, dst) in enumerate((
                (cbuf.at[0], cbuf.at[slot]),
                (wkb.at[:, pl.ds(d * HD, HD)], wkbuf.at[slot]),
                (wvb.at[:, pl.ds(d * HD, HD)], wvbuf.at[slot]),
            )):
                rdma = pltpu.make_async_remote_copy(
                    src_ref=src, dst_ref=dst,
                    send_sem=send_sems.at[slot - 1, t],
                    recv_sem=recv_sems.at[slot - 1, t],
                    device_id=(d,), device_id_type=pl.DeviceIdType.MESH,
                )
                if start:
                    rdma.start()
                rdmas.append(rdma)
        return rdmas

    rdmas = _sends(start=True)

    q_ref[...] = jnp.dot(xb, wq_ref[...],
                         preferred_element_type=F32).astype(BF)
    for h in range(HP):
        qr_ref[h] = jnp.dot(xb, wqr_ref[h],
                            preferred_element_type=F32).astype(BF)
    own_k.wait()
    own_v.wait()
    k_acc = jnp.dot(cbuf[0], wkbuf[0], preferred_element_type=F32)
    v_acc = jnp.dot(cbuf[0], wvbuf[0], preferred_element_type=F32)
    for rdma in rdmas:
        rdma.wait()
    for j in range(1, N_DEV):
        k_acc += jnp.dot(cbuf[j], wkbuf[j], preferred_element_type=F32)
        v_acc += jnp.dot(cbuf[j], wvbuf[j], preferred_element_type=F32)
    k_ref[...] = k_acc.astype(BF)
    v_ref[...] = v_acc.astype(BF)


def _attn_body(q_ref, qr_ref, k_ref, v_ref, kr_ref,
               o_ref, obuf, send_sems, recv_sems, loc_sems):
    my = lax.axis_index("i")
    h = pl.program_id(0)

    @pl.when(h == 0)
    def _():
        barrier = pltpu.get_barrier_semaphore()
        for k in range(1, N_DEV):
            pl.semaphore_signal(barrier, inc=1,
                                device_id=(lax.rem(my + k, N_DEV),),
                                device_id_type=pl.DeviceIdType.MESH)
        pl.semaphore_wait(barrier, N_DEV - 1)

    qcat = jnp.concatenate([q_ref[...], qr_ref[0]], axis=1)
    kcat = jnp.concatenate([k_ref[...], kr_ref[...]], axis=1)
    s = lax.dot_general(qcat, kcat, (((1,), (1,)), ((), ())),
                        preferred_element_type=F32)
    p = jnp.exp(s.astype(BF))
    denom = jnp.sum(p, axis=1, keepdims=True, dtype=F32)
    o = jnp.dot(p, v_ref[...], preferred_element_type=F32)
    obuf[h] = (o * (1.0 / denom)).astype(BF)

    def _hop_rdmas(hh, start):
        col = (my * HP + hh) * DH
        rdmas = [pltpu.make_async_copy(
            obuf.at[hh], o_ref.at[:, pl.ds(col, DH)], loc_sems.at[hh])]
        for k in range(1, N_DEV):
            d = lax.rem(my + k, N_DEV)
            rdmas.append(pltpu.make_async_remote_copy(
                src_ref=obuf.at[hh],
                dst_ref=o_ref.at[:, pl.ds(col, DH)],
                send_sem=send_sems.at[N_DEV - 1 - k, hh],
                recv_sem=recv_sems.at[N_DEV - 1 - k, hh],
                device_id=(d,), device_id_type=pl.DeviceIdType.MESH,
            ))
        if start:
            for rdma in rdmas:
                rdma.start()
        return rdmas

    _hop_rdmas(h, start=True)

    @pl.when(h == HP - 1)
    def _():
        for hh in range(HP):
            for rdma in _hop_rdmas(hh, start=False):
                rdma.wait()


def _proj_body(o_ref, wo_ref, out_ref):
    out_ref[...] = jnp.dot(o_ref[...], wo_ref[...].astype(BF),
                           preferred_element_type=F32)


def kernel(x, Wdkv, Wuk, Wuv, Wq, Wqr, Wkr, Wo):
    my = lax.axis_index("i")

    wq_own = lax.dynamic_slice(Wq, (0, my * HD), (D, HD))
    wq_own = (wq_own * SCALE).astype(BF)
    wqr3 = jnp.transpose((Wqr * SCALE).astype(BF).reshape(D, H, DR),
                         (1, 0, 2))
    wqr_own = lax.dynamic_slice(wqr3, (my * HP, 0, 0), (HP, D, DR))

    Q, Qr, K, V, Kr = pl.pallas_call(
        _gather_body,
        out_shape=[
            jax.ShapeDtypeStruct((S, HD), BF),
            jax.ShapeDtypeStruct((HP, S, DR), BF),
            jax.ShapeDtypeStruct((S, HD), BF),
            jax.ShapeDtypeStruct((S, HD), BF),
            jax.ShapeDtypeStruct((S, DR), BF),
        ],
        in_specs=[pl.BlockSpec(memory_space=pltpu.VMEM)] * 7,
        out_specs=[pl.BlockSpec(memory_space=pltpu.VMEM)] * 5,
        scratch_shapes=[
            pltpu.VMEM((N_DEV, S, DC), BF),
            pltpu.VMEM((DC, D), BF),
            pltpu.VMEM((DC, D), BF),
            pltpu.VMEM((N_DEV, DC, HD), BF),
            pltpu.VMEM((N_DEV, DC, HD), BF),
            pltpu.SemaphoreType.DMA((N_DEV - 1, 3)),
            pltpu.SemaphoreType.DMA((N_DEV - 1, 3)),
            pltpu.SemaphoreType.DMA((2,)),
        ],
        compiler_params=pltpu.CompilerParams(collective_id=0),
    )(x, Wdkv, Wuk, Wuv, Wkr, wq_own, wqr_own)

    O = pl.pallas_call(
        _attn_body,
        grid=(HP,),
        out_shape=jax.ShapeDtypeStruct((S, D), BF),
        in_specs=[
            pl.BlockSpec((S, DH), lambda h: (0, h)),
            pl.BlockSpec((1, S, DR), lambda h: (h, 0, 0)),
            pl.BlockSpec((S, DH), lambda h: (0, h)),
            pl.BlockSpec((S, DH), lambda h: (0, h)),
            pl.BlockSpec((S, DR), lambda h: (0, 0)),
        ],
        out_specs=pl.BlockSpec((S, D), lambda h: (0, 0)),
        scratch_shapes=[
            pltpu.VMEM((HP, S, DH), BF),
            pltpu.SemaphoreType.DMA((N_DEV - 1, HP)),
            pltpu.SemaphoreType.DMA((N_DEV - 1, HP)),
            pltpu.SemaphoreType.DMA((HP,)),
        ],
        compiler_params=pltpu.CompilerParams(collective_id=1),
    )(Q, Qr, K, V, Kr)

    out = pl.pallas_call(
        _proj_body,
        grid=(4,),
        out_shape=jax.ShapeDtypeStruct((S, D), F32),
        in_specs=[
            pl.BlockSpec((S, D), lambda j: (0, 0)),
            pl.BlockSpec((D, D // 4), lambda j: (0, j)),
        ],
        out_specs=pl.BlockSpec((S, D // 4), lambda j: (0, j)),
    )(O, Wo)

    return out[None]
